# baseline (speedup 1.0000x reference)
from functools import partial

import jax
import jax.numpy as jnp
from jax.experimental import pallas as pl

_NUM_EXPERTS = 8
_TOP_K = 2


def _router_block(x_ref, w_ref, idx_ref, wgt_ref, usage_ref, aux_ref,
                  *, nsteps, inv_ntokens):
    i = pl.program_id(0)
    xb = x_ref[...]
    w = w_ref[...]
    scores = jax.lax.dot_general(
        xb, w, (((1,), (1,)), ((), ())),
        preferred_element_type=jnp.float32)

    iota = jax.lax.broadcasted_iota(jnp.int32, scores.shape, 1)
    m1 = jnp.max(scores, axis=1, keepdims=True)
    i1 = jnp.min(jnp.where(scores == m1, iota, _NUM_EXPERTS),
                 axis=1, keepdims=True)
    masked = jnp.where(iota == i1, -jnp.inf, scores)
    m2 = jnp.max(masked, axis=1, keepdims=True)
    i2 = jnp.min(jnp.where(masked == m2, iota, _NUM_EXPERTS),
                 axis=1, keepdims=True)

    idx_ref[...] = jnp.concatenate([i1, i2], axis=1)

    e = jnp.exp(m2 - m1)
    w1 = 1.0 / (1.0 + e)
    wgt_ref[...] = jnp.concatenate([w1, 1.0 - w1], axis=1)

    p = jnp.exp(scores - m1)
    p = p / jnp.sum(p, axis=1, keepdims=True)
    part = jnp.sum(p, axis=0, keepdims=True)

    @pl.when(i == 0)
    def _():
        usage_ref[...] = part

    @pl.when(i != 0)
    def _():
        usage_ref[...] += part

    @pl.when(i == nsteps - 1)
    def _():
        u = usage_ref[...] * inv_ntokens
        aux_ref[...] = _NUM_EXPERTS * jnp.sum(u * u, axis=1, keepdims=True)


@jax.jit
def kernel(x, W):
    B, S, D = x.shape
    N = B * S
    xf = x.reshape(N, D)
    T = 2048
    nsteps = N // T

    idx, wgt, _, aux = pl.pallas_call(
        partial(_router_block, nsteps=nsteps, inv_ntokens=1.0 / N),
        grid=(nsteps,),
        in_specs=[
            pl.BlockSpec((T, D), lambda i: (i, 0)),
            pl.BlockSpec((_NUM_EXPERTS, D), lambda i: (0, 0)),
        ],
        out_specs=[
            pl.BlockSpec((T, _TOP_K), lambda i: (i, 0)),
            pl.BlockSpec((T, _TOP_K), lambda i: (i, 0)),
            pl.BlockSpec((1, _NUM_EXPERTS), lambda i: (0, 0)),
            pl.BlockSpec((1, 1), lambda i: (0, 0)),
        ],
        out_shape=[
            jax.ShapeDtypeStruct((N, _TOP_K), jnp.int32),
            jax.ShapeDtypeStruct((N, _TOP_K), jnp.float32),
            jax.ShapeDtypeStruct((1, _NUM_EXPERTS), jnp.float32),
            jax.ShapeDtypeStruct((1, 1), jnp.float32),
        ],
    )(xf, W)

    return (idx.reshape(B, S, _TOP_K),
            wgt.reshape(B, S, _TOP_K),
            aux[0, 0])


# speedup vs baseline: 2.2288x; 2.2288x over previous
from functools import partial

import jax
import jax.numpy as jnp
from jax.experimental import pallas as pl

_NUM_EXPERTS = 8
_TOP_K = 2


def _router_block(x_ref, w_ref, idx_ref, wgt_ref, usage_ref, aux_ref,
                  *, nsteps, inv_ntokens):
    i = pl.program_id(0)
    xb = x_ref[...]
    w = w_ref[...]
    scores = jax.lax.dot_general(
        w, xb, (((1,), (1,)), ((), ())),
        preferred_element_type=jnp.float32)

    iota = jax.lax.broadcasted_iota(jnp.int32, scores.shape, 0)
    m1 = jnp.max(scores, axis=0, keepdims=True)
    i1 = jnp.min(jnp.where(scores == m1, iota, _NUM_EXPERTS),
                 axis=0, keepdims=True)
    masked = jnp.where(iota == i1, -jnp.inf, scores)
    m2 = jnp.max(masked, axis=0, keepdims=True)
    i2 = jnp.min(jnp.where(masked == m2, iota, _NUM_EXPERTS),
                 axis=0, keepdims=True)

    idx_ref[...] = jnp.concatenate([i1, i2], axis=0)

    e = jnp.exp(m2 - m1)
    w1 = 1.0 / (1.0 + e)
    wgt_ref[...] = jnp.concatenate([w1, 1.0 - w1], axis=0)

    p = jnp.exp(scores - m1)
    p = p / jnp.sum(p, axis=0, keepdims=True)
    part = jnp.sum(p, axis=1, keepdims=True)

    @pl.when(i == 0)
    def _():
        usage_ref[...] = part

    @pl.when(i != 0)
    def _():
        usage_ref[...] += part

    @pl.when(i == nsteps - 1)
    def _():
        u = usage_ref[...] * inv_ntokens
        aux_ref[...] = _NUM_EXPERTS * jnp.sum(u * u, axis=0, keepdims=True)


@jax.jit
def kernel(x, W):
    B, S, D = x.shape
    N = B * S
    xf = x.reshape(N, D)
    T = 4096
    nsteps = N // T

    idx, wgt, _, aux = pl.pallas_call(
        partial(_router_block, nsteps=nsteps, inv_ntokens=1.0 / N),
        grid=(nsteps,),
        in_specs=[
            pl.BlockSpec((T, D), lambda i: (i, 0)),
            pl.BlockSpec((_NUM_EXPERTS, D), lambda i: (0, 0)),
        ],
        out_specs=[
            pl.BlockSpec((_TOP_K, T), lambda i: (0, i)),
            pl.BlockSpec((_TOP_K, T), lambda i: (0, i)),
            pl.BlockSpec((_NUM_EXPERTS, 1), lambda i: (0, 0)),
            pl.BlockSpec((1, 1), lambda i: (0, 0)),
        ],
        out_shape=[
            jax.ShapeDtypeStruct((_TOP_K, N), jnp.int32),
            jax.ShapeDtypeStruct((_TOP_K, N), jnp.float32),
            jax.ShapeDtypeStruct((_NUM_EXPERTS, 1), jnp.float32),
            jax.ShapeDtypeStruct((1, 1), jnp.float32),
        ],
    )(xf, W)

    return (idx.T.reshape(B, S, _TOP_K),
            wgt.T.reshape(B, S, _TOP_K),
            aux[0, 0])
